# baseline (speedup 1.0000x reference)
import functools

import jax
import jax.numpy as jnp
from jax import lax
from jax.experimental import pallas as pl
from jax.experimental.pallas import tpu as pltpu
from jax.experimental.pallas import tpu_sc as plsc

_VOCAB = 1000000
_DIM = 32
_BATCH = 16384
_HIST = 50
_N = _BATCH * _HIST
_NC, _NS = 2, 16
_NW = _NC * _NS
_CH = 512
_TB = _CH // 128
_CHUNKS = _N // _CH
_C_PER_H = _BATCH // _CH
_C_PER_W = _CHUNKS // _NW


def _make_gather():
    mesh = plsc.VectorSubcoreMesh(core_axis_name="c", subcore_axis_name="s")

    @functools.partial(
        pl.kernel,
        out_type=jax.ShapeDtypeStruct((_HIST, _DIM // 8, 128, 8, 128),
                                      jnp.float32),
        mesh=mesh,
        scratch_types=[
            pltpu.VMEM((_CH,), jnp.int32),
            pltpu.VMEM((_CH,), jnp.int32),
            pltpu.VMEM((_CH, _DIM), jnp.float32),
            pltpu.VMEM((_CH, _DIM), jnp.float32),
            pltpu.VMEM((_DIM // 8, _TB, 8, 129), jnp.float32),
            pltpu.VMEM((_DIM // 8, _TB, 8, 129), jnp.float32),
            pltpu.SemaphoreType.DMA,
            pltpu.SemaphoreType.DMA,
            pltpu.SemaphoreType.DMA,
        ],
        compiler_params=pltpu.CompilerParams(
            use_tc_tiling_on_sc=False, needs_layout_passes=False
        ),
    )
    def gather_kernel(idx_hbm, table_hbm, out_hbm,
                      idx0, idx1, rows0, rows1, outv0, outv1,
                      gsem0, gsem1, wsem):
        wid = lax.axis_index("s") * _NC + lax.axis_index("c")
        c0 = wid * _C_PER_W
        lane = lax.broadcasted_iota(jnp.int32, (16,), 0)
        idx_b = (idx0, idx1)
        rows_b = (rows0, rows1)
        outv_b = (outv0, outv1)
        gsem_b = (gsem0, gsem1)

        tj_lo = lane // 8
        tj_hi = tj_lo + 2
        r_pat = lane % 8

        def fetch(c, b):
            pltpu.sync_copy(idx_hbm.at[pl.ds(c * _CH, _CH)], idx_b[b])
            pltpu.async_copy(table_hbm.at[idx_b[b]], rows_b[b], gsem_b[b])

        def transpose(b):
            rows_v, out_v = rows_b[b], outv_b[b]

            @plsc.parallel_loop(0, _CH, step=1, unroll=8)
            def _rows(k):
                tbl = jnp.full((16,), k // 128, jnp.int32)
                cc = jnp.full((16,), k % 128, jnp.int32)
                for half, tj_vec in ((0, tj_lo), (1, tj_hi)):
                    v = rows_v[k, pl.ds(half * 16, 16)]
                    plsc.store_scatter(out_v, [tj_vec, tbl, r_pat, cc], v)

            plsc.subcore_barrier()

        def put(c, b):
            h = c // _C_PER_H
            tb0 = (c % _C_PER_H) * _TB
            for tj in range(_DIM // 8):
                pltpu.async_copy(outv_b[b].at[tj, :, :, pl.ds(0, 128)],
                                 out_hbm.at[h, tj, pl.ds(tb0, _TB)], wsem)

        def drain_put(b):
            for tj in range(_DIM // 8):
                pltpu.make_async_copy(
                    outv_b[b].at[tj, :, :, pl.ds(0, 128)],
                    out_hbm.at[0, tj, pl.ds(0, _TB)], wsem).wait()

        def slot(c, b, prefetch, drain_prev):
            pltpu.make_async_copy(
                table_hbm.at[idx_b[b]], rows_b[b], gsem_b[b]).wait()
            if prefetch:
                fetch(c + 1, 1 - b)
            if drain_prev:
                drain_put(1 - b)
            transpose(b)
            put(c, b)

        fetch(c0, 0)
        slot(c0, 0, prefetch=True, drain_prev=False)

        def pair(j, carry):
            c = c0 + 2 * j + 1
            slot(c, 1, prefetch=True, drain_prev=True)
            slot(c + 1, 0, prefetch=True, drain_prev=True)
            return carry

        lax.fori_loop(0, (_C_PER_W - 2) // 2, pair, 0)
        slot(c0 + _C_PER_W - 1, 1, prefetch=False, drain_prev=True)
        drain_put(1)

    return gather_kernel


_gather = _make_gather()


def kernel(inputs, table):
    idx = inputs.T.reshape(_N).astype(jnp.int32)
    out5 = _gather(idx, table)
    return out5.transpose(2, 4, 0, 1, 3).reshape(_BATCH, _HIST, _DIM)


# speedup vs baseline: 1.0886x; 1.0886x over previous
import functools

import jax
import jax.numpy as jnp
from jax import lax
from jax.experimental import pallas as pl
from jax.experimental.pallas import tpu as pltpu
from jax.experimental.pallas import tpu_sc as plsc

_VOCAB = 1000000
_DIM = 32
_BATCH = 16384
_HIST = 50
_N = _BATCH * _HIST
_NC, _NS = 2, 16
_NW = _NC * _NS
_CH = 512
_TB = _CH // 128
_CHUNKS = _N // _CH
_C_PER_H = _BATCH // _CH
_C_PER_W = _CHUNKS // _NW


def _make_gather():
    mesh = plsc.VectorSubcoreMesh(core_axis_name="c", subcore_axis_name="s")

    @functools.partial(
        pl.kernel,
        out_type=jax.ShapeDtypeStruct((_HIST, _DIM // 8, 128, 8, 128),
                                      jnp.float32),
        mesh=mesh,
        scratch_types=[
            pltpu.VMEM((_CH,), jnp.int32),
            pltpu.VMEM((_CH,), jnp.int32),
            pltpu.VMEM((_CH, _DIM), jnp.float32),
            pltpu.VMEM((_CH, _DIM), jnp.float32),
            pltpu.VMEM((_DIM // 8, _TB, 8, 129), jnp.float32),
            pltpu.VMEM((_DIM // 8, _TB, 8, 129), jnp.float32),
            pltpu.SemaphoreType.DMA,
            pltpu.SemaphoreType.DMA,
            pltpu.SemaphoreType.DMA,
        ],
        compiler_params=pltpu.CompilerParams(
            use_tc_tiling_on_sc=False, needs_layout_passes=False
        ),
    )
    def gather_kernel(idx_hbm, table4_hbm, out_hbm,
                      idx0, idx1, rows0, rows1, outv0, outv1,
                      gsem0, gsem1, wsem):
        wid = lax.axis_index("s") * _NC + lax.axis_index("c")
        c0 = wid * _C_PER_W
        lane = lax.broadcasted_iota(jnp.int32, (16,), 0)
        idx_b = (idx0, idx1)
        rows_b = (rows0, rows1)
        outv_b = (outv0, outv1)
        gsem_b = (gsem0, gsem1)

        tj_lo = lane // 8
        tj_hi = tj_lo + 2
        r_pat = lane % 8

        def fetch(c, b):
            pltpu.sync_copy(idx_hbm.at[pl.ds(c * _CH, _CH)], idx_b[b])
            pltpu.async_copy(table4_hbm.at[idx_b[b]], rows_b[b], gsem_b[b])

        def transpose(b):
            rows_v, out_v = rows_b[b], outv_b[b]

            @plsc.parallel_loop(0, _CH, step=1, unroll=8)
            def _rows(k):
                tbl = jnp.full((16,), k // 128, jnp.int32)
                cc = jnp.full((16,), k % 128, jnp.int32)
                for half, tj_vec in ((0, tj_lo), (1, tj_hi)):
                    v = rows_v[k, pl.ds(half * 16, 16)]
                    plsc.store_scatter(out_v, [tj_vec, tbl, r_pat, cc], v)

            plsc.subcore_barrier()

        def put(c, b):
            h = c // _C_PER_H
            tb0 = (c % _C_PER_H) * _TB
            for tj in range(_DIM // 8):
                pltpu.async_copy(outv_b[b].at[tj, :, :, pl.ds(0, 128)],
                                 out_hbm.at[h, tj, pl.ds(tb0, _TB)], wsem)

        def drain_put(b):
            for tj in range(_DIM // 8):
                pltpu.make_async_copy(
                    outv_b[b].at[tj, :, :, pl.ds(0, 128)],
                    out_hbm.at[0, tj, pl.ds(0, _TB)], wsem).wait()

        def slot(c, b, prefetch, drain_prev):
            pltpu.make_async_copy(
                table4_hbm.at[idx_b[b]], rows_b[b], gsem_b[b]).wait()
            if prefetch:
                fetch(c + 1, 1 - b)
            if drain_prev:
                drain_put(1 - b)
            transpose(b)
            put(c, b)

        fetch(c0, 0)
        slot(c0, 0, prefetch=True, drain_prev=False)

        def pair(j, carry):
            c = c0 + 2 * j + 1
            slot(c, 1, prefetch=True, drain_prev=True)
            slot(c + 1, 0, prefetch=True, drain_prev=True)
            return carry

        lax.fori_loop(0, (_C_PER_W - 2) // 2, pair, 0)
        slot(c0 + _C_PER_W - 1, 1, prefetch=False, drain_prev=True)
        drain_put(1)

    return gather_kernel


_gather = _make_gather()

_BV = 2048


def _linearize_body(x_ref, o_ref):
    o_ref[:, : _DIM] = x_ref[...].T


def _linearize(table):
    grid = (_VOCAB + _BV - 1) // _BV
    out = pl.pallas_call(
        _linearize_body,
        grid=(grid,),
        in_specs=[pl.BlockSpec((_DIM, _BV), lambda i: (0, i))],
        out_specs=pl.BlockSpec((_BV, 128), lambda i: (i, 0)),
        out_shape=jax.ShapeDtypeStruct((_VOCAB, 128), jnp.float32),
    )(table.T)
    return out.reshape(_VOCAB * 4, _DIM)


def kernel(inputs, table):
    idx = inputs.T.reshape(_N).astype(jnp.int32) * 4
    out5 = _gather(idx, _linearize(table))
    return out5.transpose(2, 4, 0, 1, 3).reshape(_BATCH, _HIST, _DIM)


# speedup vs baseline: 1.6488x; 1.5146x over previous
import functools

import jax
import jax.numpy as jnp
from jax import lax
from jax.experimental import pallas as pl
from jax.experimental.pallas import tpu as pltpu
from jax.experimental.pallas import tpu_sc as plsc

_VOCAB = 1000000
_DIM = 32
_BATCH = 16384
_HIST = 50
_N = _BATCH * _HIST
_NC, _NS = 2, 16
_NW = _NC * _NS
_CH = 512
_TB = _CH // 128
_CHUNKS = _N // _CH
_C_PER_H = _BATCH // _CH
_C_PER_W = _CHUNKS // _NW


def _make_gather():
    mesh = plsc.VectorSubcoreMesh(core_axis_name="c", subcore_axis_name="s")

    @functools.partial(
        pl.kernel,
        out_type=jax.ShapeDtypeStruct((_HIST, _DIM // 8, 128, 8, 128),
                                      jnp.float32),
        mesh=mesh,
        scratch_types=[
            pltpu.VMEM((_CH,), jnp.int32),
            pltpu.VMEM((_CH,), jnp.int32),
            pltpu.VMEM((_CH, _DIM), jnp.float32),
            pltpu.VMEM((_CH, _DIM), jnp.float32),
            pltpu.VMEM((_DIM // 8, _TB, 8, 129), jnp.float32),
            pltpu.VMEM((_DIM // 8, _TB, 8, 129), jnp.float32),
            pltpu.SemaphoreType.DMA,
            pltpu.SemaphoreType.DMA,
            pltpu.SemaphoreType.DMA,
        ],
        compiler_params=pltpu.CompilerParams(
            use_tc_tiling_on_sc=False, needs_layout_passes=False
        ),
    )
    def gather_kernel(idx_hbm, table4_hbm, out_hbm,
                      idx0, idx1, rows0, rows1, outv0, outv1,
                      gsem0, gsem1, wsem):
        wid = lax.axis_index("s") * _NC + lax.axis_index("c")
        c0 = wid * _C_PER_W
        lane = lax.broadcasted_iota(jnp.int32, (16,), 0)
        idx_b = (idx0, idx1)
        rows_b = (rows0, rows1)
        outv_b = (outv0, outv1)
        gsem_b = (gsem0, gsem1)

        tj_lo = lane // 8
        tj_hi = tj_lo + 2
        r_pat = lane % 8

        def fetch(c, b):
            pltpu.sync_copy(idx_hbm.at[pl.ds(c * _CH, _CH)], idx_b[b])
            pltpu.async_copy(table4_hbm.at[idx_b[b]], rows_b[b], gsem_b[b])

        def transpose(b):
            rows_v, out_v = rows_b[b], outv_b[b]

            @plsc.parallel_loop(0, _CH, step=1, unroll=8)
            def _rows(k):
                tbl = jnp.full((16,), k // 128, jnp.int32)
                cc = jnp.full((16,), k % 128, jnp.int32)
                for half, tj_vec in ((0, tj_lo), (1, tj_hi)):
                    v = rows_v[k, pl.ds(half * 16, 16)]
                    plsc.store_scatter(out_v, [tj_vec, tbl, r_pat, cc], v)

            plsc.subcore_barrier()

        def put(c, b):
            h = c // _C_PER_H
            tb0 = (c % _C_PER_H) * _TB
            for tj in range(_DIM // 8):
                pltpu.async_copy(outv_b[b].at[tj, :, :, pl.ds(0, 128)],
                                 out_hbm.at[h, tj, pl.ds(tb0, _TB)], wsem)

        def drain_put(b):
            for tj in range(_DIM // 8):
                pltpu.make_async_copy(
                    outv_b[b].at[tj, :, :, pl.ds(0, 128)],
                    out_hbm.at[0, tj, pl.ds(0, _TB)], wsem).wait()

        def slot(c, b, prefetch, drain_prev):
            pltpu.make_async_copy(
                table4_hbm.at[idx_b[b]], rows_b[b], gsem_b[b]).wait()
            if prefetch:
                fetch(c + 1, 1 - b)
            if drain_prev:
                drain_put(1 - b)
            transpose(b)
            put(c, b)

        fetch(c0, 0)
        slot(c0, 0, prefetch=True, drain_prev=False)

        def pair(j, carry):
            c = c0 + 2 * j + 1
            slot(c, 1, prefetch=True, drain_prev=True)
            slot(c + 1, 0, prefetch=True, drain_prev=True)
            return carry

        lax.fori_loop(0, (_C_PER_W - 2) // 2, pair, 0)
        slot(c0 + _C_PER_W - 1, 1, prefetch=False, drain_prev=True)
        drain_put(1)

    return gather_kernel


_gather = _make_gather()

_BV = 8192


def _linearize_body(x_ref, o_ref):
    o_ref[:, : _DIM] = x_ref[...].T


def _linearize(table):
    grid = (_VOCAB + _BV - 1) // _BV
    out = pl.pallas_call(
        _linearize_body,
        grid=(grid,),
        in_specs=[pl.BlockSpec((_DIM, _BV), lambda i: (0, i))],
        out_specs=pl.BlockSpec((_BV, 128), lambda i: (i, 0)),
        out_shape=jax.ShapeDtypeStruct((_VOCAB, 128), jnp.float32),
    )(table.T)
    return out.reshape(_VOCAB * 4, _DIM)


def kernel(inputs, table):
    idx = inputs.T.reshape(_N).astype(jnp.int32) * 4
    out5 = _gather(idx, _linearize(table))
    return out5.transpose(2, 4, 0, 1, 3).reshape(_BATCH, _HIST, _DIM)


# speedup vs baseline: 1.8003x; 1.0919x over previous
import functools

import jax
import jax.numpy as jnp
from jax import lax
from jax.experimental import pallas as pl
from jax.experimental.pallas import tpu as pltpu
from jax.experimental.pallas import tpu_sc as plsc

_VOCAB = 1000000
_DIM = 32
_BATCH = 16384
_HIST = 50
_N = _BATCH * _HIST
_NC, _NS = 2, 16
_NW = _NC * _NS
_CH = 512
_TB = _CH // 128
_CHUNKS = _N // _CH
_C_PER_H = _BATCH // _CH
_C_PER_W = _CHUNKS // _NW


def _make_gather():
    mesh = plsc.VectorSubcoreMesh(core_axis_name="c", subcore_axis_name="s")

    @functools.partial(
        pl.kernel,
        out_type=jax.ShapeDtypeStruct((_HIST, _DIM // 8, 128, 8, 128),
                                      jnp.float32),
        mesh=mesh,
        scratch_types=[
            pltpu.VMEM((_CH,), jnp.int32),
            pltpu.VMEM((_CH,), jnp.int32),
            pltpu.VMEM((_CH, _DIM), jnp.float32),
            pltpu.VMEM((_CH, _DIM), jnp.float32),
            pltpu.VMEM((_DIM // 8, _TB, 8, 129), jnp.float32),
            pltpu.VMEM((_DIM // 8, _TB, 8, 129), jnp.float32),
            pltpu.SemaphoreType.DMA,
            pltpu.SemaphoreType.DMA,
            pltpu.SemaphoreType.DMA,
        ],
        compiler_params=pltpu.CompilerParams(
            use_tc_tiling_on_sc=False, needs_layout_passes=False
        ),
    )
    def gather_kernel(idx_hbm, table4_hbm, out_hbm,
                      idx0, idx1, rows0, rows1, outv0, outv1,
                      gsem0, gsem1, wsem):
        wid = lax.axis_index("s") * _NC + lax.axis_index("c")
        c0 = wid * _C_PER_W
        lane = lax.broadcasted_iota(jnp.int32, (16,), 0)
        idx_b = (idx0, idx1)
        rows_b = (rows0, rows1)
        outv_b = (outv0, outv1)
        gsem_b = (gsem0, gsem1)

        tj_lo = lane // 8
        tj_hi = tj_lo + 2
        r_pat = lane % 8

        def fetch(c, b):
            pltpu.sync_copy(idx_hbm.at[pl.ds(c * _CH, _CH)], idx_b[b])
            pltpu.async_copy(table4_hbm.at[idx_b[b]], rows_b[b], gsem_b[b])

        def transpose(b):
            rows_v, out_v = rows_b[b], outv_b[b]

            @plsc.parallel_loop(0, _CH, step=1, unroll=8)
            def _rows(k):
                tbl = jnp.full((16,), k // 128, jnp.int32)
                cc = jnp.full((16,), k % 128, jnp.int32)
                for half, tj_vec in ((0, tj_lo), (1, tj_hi)):
                    v = rows_v[k, pl.ds(half * 16, 16)]
                    plsc.store_scatter(out_v, [tj_vec, tbl, r_pat, cc], v)

            plsc.subcore_barrier()

        def put(c, b):
            h = c // _C_PER_H
            tb0 = (c % _C_PER_H) * _TB
            for tj in range(_DIM // 8):
                pltpu.async_copy(outv_b[b].at[tj, :, :, pl.ds(0, 128)],
                                 out_hbm.at[h, tj, pl.ds(tb0, _TB)], wsem)

        def drain_put(b):
            for tj in range(_DIM // 8):
                pltpu.make_async_copy(
                    outv_b[b].at[tj, :, :, pl.ds(0, 128)],
                    out_hbm.at[0, tj, pl.ds(0, _TB)], wsem).wait()

        def slot(c, b, prefetch, drain_prev):
            pltpu.make_async_copy(
                table4_hbm.at[idx_b[b]], rows_b[b], gsem_b[b]).wait()
            if prefetch:
                fetch(c + 1, 1 - b)
            if drain_prev:
                drain_put(1 - b)
            transpose(b)
            put(c, b)

        fetch(c0, 0)
        slot(c0, 0, prefetch=True, drain_prev=False)

        def pair(j, carry):
            c = c0 + 2 * j + 1
            slot(c, 1, prefetch=True, drain_prev=True)
            slot(c + 1, 0, prefetch=True, drain_prev=True)
            return carry

        lax.fori_loop(0, (_C_PER_W - 2) // 2, pair, 0)
        slot(c0 + _C_PER_W - 1, 1, prefetch=False, drain_prev=True)
        drain_put(1)

    return gather_kernel


_gather = _make_gather()

_BV = 16384


def _linearize_body(x_ref, o_ref):
    o_ref[:, : _DIM] = x_ref[...].T


def _linearize(table):
    grid = (_VOCAB + _BV - 1) // _BV
    out = pl.pallas_call(
        _linearize_body,
        grid=(grid,),
        in_specs=[pl.BlockSpec((_DIM, _BV), lambda i: (0, i))],
        out_specs=pl.BlockSpec((_BV, 128), lambda i: (i, 0)),
        out_shape=jax.ShapeDtypeStruct((_VOCAB, 128), jnp.float32),
    )(table.T)
    return out.reshape(_VOCAB * 4, _DIM)


def kernel(inputs, table):
    idx = inputs.T.reshape(_N).astype(jnp.int32) * 4
    out5 = _gather(idx, _linearize(table))
    return out5.transpose(2, 4, 0, 1, 3).reshape(_BATCH, _HIST, _DIM)
